# baseline (speedup 1.0000x reference)
import functools

import jax
import jax.numpy as jnp
import numpy as np
from jax import lax
from jax.experimental import pallas as pl
from jax.experimental.pallas import tpu as pltpu
from jax.experimental.pallas import tpu_sc as plsc

IRR = [(128, 0, 1), (64, 1, 3), (32, 2, 5)]
CTOT = sum(m * d for m, _, d in IRR)
NSEG = 512
EPSV = 1e-05
RBLK = 5000
NSTAT = 16

NTILE = 32
ROWS_PER_TILE = 1568
CHUNK = 112
NCHUNK = ROWS_PER_TILE // CHUNK
NPAD_SC = NTILE * ROWS_PER_TILE
SEGPAD = 520

_PREC = jax.lax.Precision.DEFAULT


def _build_consts():
    P = np.zeros((CTOT, NSTAT), np.float32)
    Q = np.zeros((128, NSTAT), np.float32)
    Em = np.zeros((NSTAT, CTOT), np.float32)
    Es = np.zeros((NSTAT, CTOT), np.float32)
    cscale = np.zeros((CTOT,), np.float32)
    widx = np.zeros((CTOT,), np.int32)
    col = 0
    mulbase = 0
    stat = 0
    for (mul, l, d) in IRR:
        for m in range(mul):
            for k in range(d):
                c = col + m * d + k
                P[c, stat + k] = 1.0
                Em[stat + k, c] = 1.0
                widx[c] = mulbase + m
                if l == 0:
                    Es[9, c] = 1.0
                else:
                    cscale[c] = 1.0
        col += mul * d
        mulbase += mul
        stat += d
    Q[:, 9] = 1.0
    return P, Q, Em, Es, cscale, widx


_P, _Q, _EM, _ES, _CSCALE, _WIDX = _build_consts()


def _k_rowstats(xb_ref, p_ref, q_ref, rs_ref):
    xb = xb_ref[...]
    rs = jax.lax.dot_general(xb, p_ref[...], (((1,), (0,)), ((), ())),
                             precision=_PREC,
                             preferred_element_type=jnp.float32)
    xs = xb[:, :128]
    rs = rs + jax.lax.dot_general(xs * xs, q_ref[...],
                                  (((1,), (0,)), ((), ())),
                                  precision=_PREC,
                                  preferred_element_type=jnp.float32)
    lane = jax.lax.broadcasted_iota(jnp.int32, (RBLK, NSTAT), 1)
    rs_ref[...] = rs + (lane == 10).astype(jnp.float32)


def _k_scatter(rs_hbm, bat_hbm, out_hbm, rs_v, bat_v, acc):
    c = lax.axis_index("c")
    s = lax.axis_index("s")
    w = c * 16 + s
    base = w * ROWS_PER_TILE
    pltpu.sync_copy(rs_hbm.at[pl.ds(base * NSTAT, ROWS_PER_TILE * NSTAT)],
                    rs_v)
    pltpu.sync_copy(bat_hbm.at[pl.ds(base, ROWS_PER_TILE)], bat_v)
    zv = jnp.zeros((NSTAT,), jnp.float32)

    def zbody(i, _):
        acc[pl.ds(i * NSTAT, NSTAT)] = zv
        return ()

    lax.fori_loop(0, SEGPAD, zbody, ())

    def body(g, _):
        b16 = bat_v[pl.ds(g * 16, 16)]
        for jj in range(16):
            seg = b16[jj]
            row = rs_v[pl.ds(g * 256 + jj * NSTAT, NSTAT)]
            plsc.addupdate(acc.at[pl.ds(seg * NSTAT, NSTAT)], row)
        return ()

    lax.fori_loop(0, ROWS_PER_TILE // 16, body, ())
    pltpu.sync_copy(acc, out_hbm.at[w])


def _finalize(s):
    cnt = s[:, 10:11]
    n = jnp.maximum(cnt, 1.0)
    lane0 = jax.lax.broadcasted_iota(jnp.int32, (NSEG, NSTAT), 1)
    dv = jnp.where(lane0 < 1, 128.0,
                   jnp.where(lane0 < 4, 64.0, jnp.where(lane0 < 9, 32.0, 1.0)))
    mean_all = s / (dv * n)
    s0 = s[:, 0:1]
    q0 = s[:, 9:10]
    norm = (q0 - s0 * s0 / (128.0 * n)) / (128.0 * n)
    inv = 1.0 / (jnp.sqrt(jnp.maximum(norm, 0.0)) + EPSV)
    lane = jax.lax.broadcasted_iota(jnp.int32, (NSEG, NSTAT), 1)
    return jnp.where(lane < 9, mean_all, jnp.where(lane == 9, inv, 0.0))


def _k_apply(xb_ref, bat_ref, stats_ref, em_ref, es_ref, w2_ref, out_ref,
             params_ref):
    @pl.when(pl.program_id(0) == 0)
    def _():
        s2 = jnp.sum(stats_ref[:, :NSEG, :], axis=0)
        params_ref[...] = _finalize(s2)

    bat = bat_ref[0, 0, :]
    seg = jax.lax.broadcasted_iota(jnp.int32, (RBLK, NSEG), 1)
    oh = (seg == bat[:, None]).astype(jnp.float32)
    g = jax.lax.dot_general(oh, params_ref[...], (((1,), (0,)), ((), ())),
                            precision=_PREC,
                            preferred_element_type=jnp.float32)
    meanc = jax.lax.dot_general(g, em_ref[...], (((1,), (0,)), ((), ())),
                                precision=_PREC,
                                preferred_element_type=jnp.float32)
    scalec = jax.lax.dot_general(g, es_ref[...], (((1,), (0,)), ((), ())),
                                 precision=_PREC,
                                 preferred_element_type=jnp.float32)
    scalec = scalec + w2_ref[2:3, :]
    out_ref[...] = ((xb_ref[...] - meanc) * scalec * w2_ref[0:1, :]
                    + w2_ref[1:2, :])


def _make_scatter_call():
    mesh = plsc.VectorSubcoreMesh(core_axis_name="c", subcore_axis_name="s")
    return functools.partial(
        pl.kernel,
        mesh=mesh,
        out_type=jax.ShapeDtypeStruct((NTILE, SEGPAD * NSTAT), jnp.float32),
        scratch_types=[
            pltpu.VMEM((ROWS_PER_TILE * NSTAT,), jnp.float32),
            pltpu.VMEM((ROWS_PER_TILE,), jnp.int32),
            pltpu.VMEM((SEGPAD * NSTAT,), jnp.float32),
        ],
    )(_k_scatter)


@jax.jit
def kernel(x, batch, weight, bias):
    n = x.shape[0]
    nblk = (n + RBLK - 1) // RBLK
    npad = nblk * RBLK
    batch = batch.astype(jnp.int32)
    if npad == n:
        xpad = x
        batpad = batch.reshape(nblk, 1, RBLK)
    else:
        xpad = jnp.pad(x, ((0, npad - n), (0, 0)))
        batpad = jnp.pad(batch, (0, npad - n),
                         constant_values=NSEG).reshape(nblk, 1, RBLK)
    wcol = weight[jnp.asarray(_WIDX)]
    bcol = jnp.concatenate([bias, jnp.zeros((CTOT - bias.shape[0],),
                                            jnp.float32)])
    w2 = jnp.zeros((8, CTOT), jnp.float32)
    w2 = w2.at[0].set(wcol).at[1].set(bcol).at[2].set(jnp.asarray(_CSCALE))

    cmap = lambda i: (0, 0)
    rs = pl.pallas_call(
        _k_rowstats,
        grid=(nblk,),
        in_specs=[
            pl.BlockSpec((RBLK, CTOT), lambda i: (i, 0)),
            pl.BlockSpec((CTOT, NSTAT), cmap),
            pl.BlockSpec((128, NSTAT), cmap),
        ],
        out_specs=pl.BlockSpec((RBLK, NSTAT), lambda i: (i, 0)),
        out_shape=jax.ShapeDtypeStruct((n, NSTAT), jnp.float32),
    )(xpad, jnp.asarray(_P), jnp.asarray(_Q))

    rsp = jnp.pad(rs, ((0, NPAD_SC - n), (0, 0))).reshape(-1)
    batp = jnp.pad(batch, (0, NPAD_SC - n), constant_values=NSEG)
    stats2 = _make_scatter_call()(rsp, batp)
    stats2 = stats2.reshape(NTILE, SEGPAD, NSTAT)

    out = pl.pallas_call(
        _k_apply,
        grid=(nblk,),
        in_specs=[
            pl.BlockSpec((RBLK, CTOT), lambda i: (i, 0)),
            pl.BlockSpec((1, 1, RBLK), lambda i: (i, 0, 0)),
            pl.BlockSpec((NTILE, SEGPAD, NSTAT), lambda i: (0, 0, 0)),
            pl.BlockSpec((NSTAT, CTOT), cmap),
            pl.BlockSpec((NSTAT, CTOT), cmap),
            pl.BlockSpec((8, CTOT), cmap),
        ],
        out_specs=pl.BlockSpec((RBLK, CTOT), lambda i: (i, 0)),
        out_shape=jax.ShapeDtypeStruct((npad, CTOT), jnp.float32),
        scratch_shapes=[pltpu.VMEM((NSEG, NSTAT), jnp.float32)],
    )(xpad, batpad, stats2, jnp.asarray(_EM), jnp.asarray(_ES), w2)
    return out[:n]
